# baseline (speedup 1.0000x reference)
import jax
import jax.numpy as jnp
from jax.experimental import pallas as pl
from jax.experimental.pallas import tpu as pltpu

_B = 4
_S = 8192
_D = 2048
_N = 64
_STRIDE = 128
_ROWS = _B * _N
_C = 8
_RPC = _ROWS // _C
_HPB = _N // _RPC


def _copy(x_hbm, out_hbm, sem, c):
    b, h = divmod(c, _HPB)
    return pltpu.make_async_copy(
        x_hbm.at[b, pl.ds(h * _RPC, _RPC), 0, :],
        out_hbm.at[pl.ds(c * _RPC, _RPC)],
        sem.at[c],
    )


def _gather_body(x_hbm, out_hbm, sem):
    for c in range(_C):
        _copy(x_hbm, out_hbm, sem, c).start()
    for c in range(_C):
        _copy(x_hbm, out_hbm, sem, c).wait()


def kernel(x):
    x4 = x.reshape(_B, _N, _STRIDE, _D)
    out = pl.pallas_call(
        _gather_body,
        in_specs=[pl.BlockSpec(memory_space=pl.ANY)],
        out_specs=pl.BlockSpec(memory_space=pl.ANY),
        out_shape=jax.ShapeDtypeStruct((_ROWS, _D), jnp.float32),
        scratch_shapes=[pltpu.SemaphoreType.DMA((_C,))],
    )(x4)
    return out.reshape(_B, _N, _D)


# speedup vs baseline: 23.3649x; 23.3649x over previous
import jax
import jax.numpy as jnp
from jax.experimental import pallas as pl
from jax.experimental.pallas import tpu as pltpu

_B = 4
_S = 8192
_D = 2048
_N = 64
_STRIDE = 128
_ROWS = _B * _N
_C = 16
_RPC = _ROWS // _C
_HPB = _N // _RPC


def _read(x_hbm, buf, rsem, c):
    b, h = divmod(c, _HPB)
    return pltpu.make_async_copy(
        x_hbm.at[b, pl.ds(h * _RPC, _RPC), 0, :],
        buf.at[pl.ds(c * _RPC, _RPC)],
        rsem.at[c],
    )


def _write(buf, out_hbm, wsem, c):
    return pltpu.make_async_copy(
        buf.at[pl.ds(c * _RPC, _RPC)],
        out_hbm.at[pl.ds(c * _RPC, _RPC)],
        wsem.at[c],
    )


def _gather_body(x_hbm, out_hbm, buf, rsem, wsem):
    for c in range(_C):
        _read(x_hbm, buf, rsem, c).start()
    for c in range(_C):
        _read(x_hbm, buf, rsem, c).wait()
        _write(buf, out_hbm, wsem, c).start()
    for c in range(_C):
        _write(buf, out_hbm, wsem, c).wait()


def kernel(x):
    x4 = x.reshape(_B, _N, _STRIDE, _D)
    out = pl.pallas_call(
        _gather_body,
        in_specs=[pl.BlockSpec(memory_space=pl.ANY)],
        out_specs=pl.BlockSpec(memory_space=pl.ANY),
        out_shape=jax.ShapeDtypeStruct((_ROWS, _D), jnp.float32),
        scratch_shapes=[
            pltpu.VMEM((_ROWS, _D), jnp.float32),
            pltpu.SemaphoreType.DMA((_C,)),
            pltpu.SemaphoreType.DMA((_C,)),
        ],
    )(x4)
    return out.reshape(_B, _N, _D)


# speedup vs baseline: 24.6421x; 1.0547x over previous
import jax
import jax.numpy as jnp
from jax.experimental import pallas as pl
from jax.experimental.pallas import tpu as pltpu

_B = 4
_S = 8192
_D = 2048
_N = 64
_STRIDE = 128
_ROWS = _B * _N
_C = 4
_RPC = _ROWS // _C
_HPB = _N // _RPC


def _read(x_hbm, buf, rsem, c):
    b, h = divmod(c, _HPB)
    return pltpu.make_async_copy(
        x_hbm.at[b, pl.ds(h * _RPC, _RPC), 0, :],
        buf.at[pl.ds(c * _RPC, _RPC)],
        rsem.at[c],
    )


def _write(buf, out_hbm, wsem, c):
    return pltpu.make_async_copy(
        buf.at[pl.ds(c * _RPC, _RPC)],
        out_hbm.at[pl.ds(c * _RPC, _RPC)],
        wsem.at[c],
    )


def _gather_body(x_hbm, out_hbm, buf, rsem, wsem):
    for c in range(_C):
        _read(x_hbm, buf, rsem, c).start()
    for c in range(_C):
        _read(x_hbm, buf, rsem, c).wait()
        _write(buf, out_hbm, wsem, c).start()
    for c in range(_C):
        _write(buf, out_hbm, wsem, c).wait()


def kernel(x):
    x4 = x.reshape(_B, _N, _STRIDE, _D)
    out = pl.pallas_call(
        _gather_body,
        in_specs=[pl.BlockSpec(memory_space=pl.ANY)],
        out_specs=pl.BlockSpec(memory_space=pl.ANY),
        out_shape=jax.ShapeDtypeStruct((_ROWS, _D), jnp.float32),
        scratch_shapes=[
            pltpu.VMEM((_ROWS, _D), jnp.float32),
            pltpu.SemaphoreType.DMA((_C,)),
            pltpu.SemaphoreType.DMA((_C,)),
        ],
    )(x4)
    return out.reshape(_B, _N, _D)
